# baseline (speedup 1.0000x reference)
import jax
import jax.numpy as jnp
from jax import lax
from jax.experimental import pallas as pl
from jax.experimental.pallas import tpu as pltpu
from jax.experimental.pallas import tpu_sc as plsc

B = 16384
D = 64
NC = 2
NS = 16
NW = NC * NS
BPW = B // NW

NU = 1000000
NR = 100000
NCOLS_U = (NU + 127) // 128
NCOLS_S = (NR + 127) // 128
ITROWS = B + 16
MCAP = 2048
RCAP = 768


def _iota16():
    return lax.broadcasted_iota(jnp.int32, (16,), 0)


def _splat(x):
    return jnp.full((16,), x, jnp.int32)


def _k1_body(uid_h, rid_h, ing_h, utT_h, rtT_h, itT_h, tu_h, tr_h, tg_h,
             it0_h, it1_h, it2_h,
             idxchunk, mk, mi, colbuf0, colbuf1, rowsbuf, sidx,
             gsem0, gsem1, ssem):
    wid = lax.axis_index("s") * NC + lax.axis_index("c")

    def process(tbl, tail, idxh, ith, ncols):
        full = ncols - 1
        lo = wid * full // NW
        hi = (wid + 1) * full // NW
        hi_scan = jnp.where(wid == NW - 1, hi + 1, hi)

        def init_body(i, c):
            plsc.store_scatter(sidx, [_splat(i // 8), (i % 8) * 16 + _iota16()],
                               _splat(B) + _iota16(),
                               mask=_splat(1) > 0)
            return c
        lax.fori_loop(0, RCAP // 16, init_body, 0)

        def scan_v(i, off, ch):
            v = idxchunk[pl.ds(i * 16, 16)]
            tc = lax.shift_right_logical(v, 7)
            m = (tc >= lo) & (tc < hi_scan)
            cntv = plsc.all_reduce_population_count(m)
            k = _splat(ch * 512) + _iota16() + i * 16
            o = jnp.minimum(off, MCAP - 16)
            plsc.store_compressed(mk.at[pl.ds(o, 16)], k, mask=m)
            plsc.store_compressed(mi.at[pl.ds(o, 16)], v, mask=m)
            return off + cntv[0]

        def scan_chunk(ch, off):
            pltpu.sync_copy(idxh.at[pl.ds(ch * 512, 512)], idxchunk)
            return lax.fori_loop(0, 32, lambda i, o: scan_v(i, o, 0) if False
                                 else scan_v(i, o, ch), off)

        M = lax.fori_loop(0, B // 512, scan_chunk, 0)
        M = jnp.minimum(M, MCAP)
        Mv = (M + 15) // 16

        def src_slice(c):
            start = pl.multiple_of(c * 128, 128)
            return tbl.at[pl.ds(0, 64), pl.ds(start, 128)]

        def extract_from(colbuf, c, slot):
            def mv_body(v, slot):
                valid = (_iota16() + v * 16) < M
                iv = mi[pl.ds(v * 16, 16)]
                kv = mk[pl.ds(v * 16, 16)]
                tc = lax.shift_right_logical(iv, 7)
                msk = valid & (tc == c)
                cnt = plsc.all_reduce_population_count(msk)[0]

                @pl.when(cnt > 0)
                def _():
                    lv = iv & 127
                    rank = plsc.cumsum(jnp.where(msk, 1, 0)) - 1
                    slots = jnp.minimum(slot + rank, RCAP - 1)
                    for j in range(D):
                        vals = plsc.load_gather(colbuf, [_splat(j), lv])
                        plsc.store_scatter(rowsbuf, [slots, _splat(j)],
                                           vals, mask=msk)
                    plsc.store_scatter(sidx, [slots // 128, slots % 128],
                                       kv, mask=msk)

                return slot + cnt
            return lax.fori_loop(0, Mv, mv_body, slot)

        pltpu.async_copy(src_slice(lo), colbuf0, gsem0)
        n = hi - lo
        npairs = (n + 1) // 2

        def pair_body(p, slot):
            c0 = lo + p * 2
            c1c = jnp.minimum(c0 + 1, hi - 1)
            c2c = jnp.minimum(c0 + 2, hi - 1)
            pltpu.async_copy(src_slice(c1c), colbuf1, gsem1)
            pltpu.make_async_copy(src_slice(c0), colbuf0, gsem0).wait()
            slot = extract_from(colbuf0, c0, slot)
            pltpu.async_copy(src_slice(c2c), colbuf0, gsem0)
            pltpu.make_async_copy(src_slice(c1c), colbuf1, gsem1).wait()
            slot = extract_from(colbuf1, c0 + 1, slot)
            return slot

        slot = lax.fori_loop(0, npairs, pair_body, 0)
        pltpu.make_async_copy(src_slice(jnp.minimum(hi - 1, full - 1)),
                              colbuf0, gsem0).wait()
        pltpu.sync_copy(tail, colbuf0)
        slot = extract_from(colbuf0, full, slot)
        cps = [pltpu.async_copy(rowsbuf.at[pl.ds(128 * b_, 128)],
                                ith.at[sidx.at[b_]], ssem)
               for b_ in range(RCAP // 128)]
        for cp in cps:
            cp.wait()

    process(utT_h, tu_h, uid_h, it0_h, NCOLS_U)
    process(rtT_h, tr_h, rid_h, it1_h, NCOLS_S)
    process(itT_h, tg_h, ing_h, it2_h, NCOLS_S)


def _k2_body(it0_h, it1_h, it2_h, out_h, bigbuf, tilebuf):
    wid = lax.axis_index("s") * NC + lax.axis_index("c")
    base = pl.multiple_of(wid * BPW, BPW)
    for t, ith in enumerate((it0_h, it1_h, it2_h)):
        pltpu.sync_copy(ith.at[pl.ds(base, BPW)], bigbuf)

        def tile_body(q, c):
            jr = q // 4
            kc = q % 4
            for j in range(8):
                f = 8 * jr + j
                for b_ in range(8):
                    kv = _iota16() + (kc * 128 + b_ * 16)
                    vals = plsc.load_gather(bigbuf, [kv, f + _splat(0)])
                    plsc.store_scatter(tilebuf,
                                       [_splat(j), _iota16() + b_ * 16], vals)
            rs = pl.multiple_of(64 * t + 8 * jr, 8)
            cs = pl.multiple_of(base + 128 * kc, 128)
            pltpu.sync_copy(tilebuf, out_h.at[pl.ds(rs, 8), pl.ds(cs, 128)])
            return c
        lax.fori_loop(0, 32, tile_body, 0)


def kernel(uid, rid, ing, user_table, recipe_table, ingredient_table):
    tails = []
    for tb, ncols in ((user_table, NCOLS_U), (recipe_table, NCOLS_S),
                      (ingredient_table, NCOLS_S)):
        tail = tb[(ncols - 1) * 128:]
        tailp = jnp.pad(tail, ((0, 128 - tail.shape[0]), (0, 0)))
        tails.append(tailp.T)

    mesh = plsc.VectorSubcoreMesh(core_axis_name="c", subcore_axis_name="s")
    it_shape = jax.ShapeDtypeStruct((ITROWS, 128), jnp.float32)
    k1 = pl.kernel(
        _k1_body,
        mesh=mesh,
        compiler_params=pltpu.CompilerParams(needs_layout_passes=False),
        out_type=(it_shape, it_shape, it_shape),
        scratch_types=[
            pltpu.VMEM((512,), jnp.int32),
            pltpu.VMEM((MCAP,), jnp.int32),
            pltpu.VMEM((MCAP,), jnp.int32),
            pltpu.VMEM((64, 128), jnp.float32),
            pltpu.VMEM((64, 128), jnp.float32),
            pltpu.VMEM((RCAP, 128), jnp.float32),
            pltpu.VMEM((RCAP // 128, 128), jnp.int32),
            pltpu.SemaphoreType.DMA,
            pltpu.SemaphoreType.DMA,
            pltpu.SemaphoreType.DMA,
        ],
    )
    it0, it1, it2 = k1(uid, rid, ing, user_table.T, recipe_table.T,
                       ingredient_table.T, *tails)

    k2 = pl.kernel(
        _k2_body,
        mesh=mesh,
        compiler_params=pltpu.CompilerParams(needs_layout_passes=False),
        out_type=jax.ShapeDtypeStruct((3 * D, B), jnp.float32),
        scratch_types=[
            pltpu.VMEM((BPW, 128), jnp.float32),
            pltpu.VMEM((8, 128), jnp.float32),
        ],
    )
    outT = k2(it0, it1, it2)
    return outT.T


# speedup vs baseline: 1.0502x; 1.0502x over previous
import jax
import jax.numpy as jnp
from jax import lax
from jax.experimental import pallas as pl
from jax.experimental.pallas import tpu as pltpu
from jax.experimental.pallas import tpu_sc as plsc

B = 16384
D = 64
NC = 2
NS = 16
NW = NC * NS
BPW = B // NW

NU = 1000000
NR = 100000
NCOLS_U = (NU + 127) // 128
NCOLS_S = (NR + 127) // 128
ITROWS = B + 16
MCAP = 1024
RCAP = 640


def _iota16():
    return lax.broadcasted_iota(jnp.int32, (16,), 0)


def _splat(x):
    return jnp.full((16,), x, jnp.int32)


def _k1_body(uid_h, rid_h, ing_h, utT_h, rtT_h, itT_h, tu_h, tr_h, tg_h,
             it0_h, it1_h, it2_h,
             idxfull, mk, mi, colbuf0, colbuf1, rowsbuf, sidx,
             gsem0, gsem1, ssem):
    wid = lax.axis_index("s") * NC + lax.axis_index("c")

    def process(tbl, tail, idxh, ith, ncols):
        full = ncols - 1
        lo = wid * full // NW
        hi = (wid + 1) * full // NW
        hi_scan = jnp.where(wid == NW - 1, hi + 1, hi)

        def init_body(i, c):
            plsc.store_scatter(sidx, [_splat(i // 8), (i % 8) * 16 + _iota16()],
                               _splat(B) + _iota16(),
                               mask=_splat(1) > 0)
            return c
        lax.fori_loop(0, RCAP // 16, init_body, 0)

        pltpu.sync_copy(idxh, idxfull)

        def scan_v(i, off):
            v = idxfull[pl.ds(i * 16, 16)]
            tc = lax.shift_right_logical(v, 7)
            m = (tc >= lo) & (tc < hi_scan)
            cntv = plsc.all_reduce_population_count(m)
            k = _iota16() + i * 16
            o = jnp.minimum(off, MCAP - 16)
            plsc.store_compressed(mk.at[pl.ds(o, 16)], k, mask=m)
            plsc.store_compressed(mi.at[pl.ds(o, 16)], v, mask=m)
            return off + cntv[0]

        M = lax.fori_loop(0, B // 16, scan_v, 0)
        M = jnp.minimum(M, MCAP)
        Mv = (M + 15) // 16

        def src_slice(c):
            start = pl.multiple_of(c * 128, 128)
            return tbl.at[pl.ds(0, 64), pl.ds(start, 128)]

        def extract_from(colbuf, c, slot):
            def mv_body(v, slot):
                valid = (_iota16() + v * 16) < M
                iv = mi[pl.ds(v * 16, 16)]
                kv = mk[pl.ds(v * 16, 16)]
                tc = lax.shift_right_logical(iv, 7)
                msk = valid & (tc == c)
                cnt = plsc.all_reduce_population_count(msk)[0]

                @pl.when(cnt > 0)
                def _():
                    lv = iv & 127
                    rank = plsc.cumsum(jnp.where(msk, 1, 0)) - 1
                    slots = jnp.minimum(slot + rank, RCAP - 1)

                    def jloop(j, jv):
                        vals = plsc.load_gather(colbuf, [jv, lv])
                        plsc.store_scatter(rowsbuf, [slots, jv],
                                           vals, mask=msk)
                        return jv + 1
                    lax.fori_loop(0, D, jloop, _splat(0))
                    plsc.store_scatter(sidx, [slots // 128, slots % 128],
                                       kv, mask=msk)

                return slot + cnt
            return lax.fori_loop(0, Mv, mv_body, slot)

        pltpu.async_copy(src_slice(lo), colbuf0, gsem0)
        n = hi - lo
        npairs = (n + 1) // 2

        def pair_body(p, slot):
            c0 = lo + p * 2
            c1c = jnp.minimum(c0 + 1, hi - 1)
            c2c = jnp.minimum(c0 + 2, hi - 1)
            pltpu.async_copy(src_slice(c1c), colbuf1, gsem1)
            pltpu.make_async_copy(src_slice(c0), colbuf0, gsem0).wait()
            slot = extract_from(colbuf0, c0, slot)
            pltpu.async_copy(src_slice(c2c), colbuf0, gsem0)
            pltpu.make_async_copy(src_slice(c1c), colbuf1, gsem1).wait()
            slot = extract_from(colbuf1, c0 + 1, slot)
            return slot

        slot = lax.fori_loop(0, npairs, pair_body, 0)
        pltpu.make_async_copy(src_slice(jnp.minimum(hi - 1, full - 1)),
                              colbuf0, gsem0).wait()
        pltpu.sync_copy(tail, colbuf0)
        slot = extract_from(colbuf0, full, slot)
        cps = [pltpu.async_copy(rowsbuf.at[pl.ds(128 * b_, 128)],
                                ith.at[sidx.at[b_]], ssem)
               for b_ in range(RCAP // 128)]
        for cp in cps:
            cp.wait()

    process(utT_h, tu_h, uid_h, it0_h, NCOLS_U)
    process(rtT_h, tr_h, rid_h, it1_h, NCOLS_S)
    process(itT_h, tg_h, ing_h, it2_h, NCOLS_S)


def _k2_body(it0_h, it1_h, it2_h, out_h, bigbuf, tilebuf0, tilebuf1,
             wsem0, wsem1):
    wid = lax.axis_index("s") * NC + lax.axis_index("c")
    base = pl.multiple_of(wid * BPW, BPW)
    drain = out_h.at[pl.ds(0, 8), pl.ds(0, 128)]
    for t, ith in enumerate((it0_h, it1_h, it2_h)):
        pltpu.sync_copy(ith.at[pl.ds(base, BPW)], bigbuf)

        def pair_body(p, c):
            for par, tb, sm in ((0, tilebuf0, wsem0), (1, tilebuf1, wsem1)):
                q = p * 2 + par
                jr = q // 4
                kc = q % 4

                @pl.when(p > 0)
                def _():
                    pltpu.make_async_copy(tb, drain, sm).wait()
                for j in range(8):
                    f = 8 * jr + j
                    for b_ in range(8):
                        kv = _iota16() + (kc * 128 + b_ * 16)
                        vals = plsc.load_gather(bigbuf, [kv, f + _splat(0)])
                        plsc.store_scatter(tb,
                                           [_splat(j), _iota16() + b_ * 16],
                                           vals)
                rs = pl.multiple_of(64 * t + 8 * jr, 8)
                cs = pl.multiple_of(base + 128 * kc, 128)
                pltpu.async_copy(tb, out_h.at[pl.ds(rs, 8), pl.ds(cs, 128)],
                                 sm)
            return c
        lax.fori_loop(0, 16, pair_body, 0)
        pltpu.make_async_copy(tilebuf0, drain, wsem0).wait()
        pltpu.make_async_copy(tilebuf1, drain, wsem1).wait()


def kernel(uid, rid, ing, user_table, recipe_table, ingredient_table):
    tails = []
    for tb, ncols in ((user_table, NCOLS_U), (recipe_table, NCOLS_S),
                      (ingredient_table, NCOLS_S)):
        tail = tb[(ncols - 1) * 128:]
        tailp = jnp.pad(tail, ((0, 128 - tail.shape[0]), (0, 0)))
        tails.append(tailp.T)

    mesh = plsc.VectorSubcoreMesh(core_axis_name="c", subcore_axis_name="s")
    it_shape = jax.ShapeDtypeStruct((ITROWS, 128), jnp.float32)
    k1 = pl.kernel(
        _k1_body,
        mesh=mesh,
        compiler_params=pltpu.CompilerParams(needs_layout_passes=False),
        out_type=(it_shape, it_shape, it_shape),
        scratch_types=[
            pltpu.VMEM((B,), jnp.int32),
            pltpu.VMEM((MCAP,), jnp.int32),
            pltpu.VMEM((MCAP,), jnp.int32),
            pltpu.VMEM((64, 128), jnp.float32),
            pltpu.VMEM((64, 128), jnp.float32),
            pltpu.VMEM((RCAP, 128), jnp.float32),
            pltpu.VMEM((RCAP // 128, 128), jnp.int32),
            pltpu.SemaphoreType.DMA,
            pltpu.SemaphoreType.DMA,
            pltpu.SemaphoreType.DMA,
        ],
    )
    it0, it1, it2 = k1(uid, rid, ing, user_table.T, recipe_table.T,
                       ingredient_table.T, *tails)

    k2 = pl.kernel(
        _k2_body,
        mesh=mesh,
        compiler_params=pltpu.CompilerParams(needs_layout_passes=False),
        out_type=jax.ShapeDtypeStruct((3 * D, B), jnp.float32),
        scratch_types=[
            pltpu.VMEM((BPW, 128), jnp.float32),
            pltpu.VMEM((8, 128), jnp.float32),
            pltpu.VMEM((8, 128), jnp.float32),
            pltpu.SemaphoreType.DMA,
            pltpu.SemaphoreType.DMA,
        ],
    )
    outT = k2(it0, it1, it2)
    return outT.T


# speedup vs baseline: 1.0990x; 1.0464x over previous
import jax
import jax.numpy as jnp
from jax import lax
from jax.experimental import pallas as pl
from jax.experimental.pallas import tpu as pltpu
from jax.experimental.pallas import tpu_sc as plsc

B = 16384
D = 64
NC = 2
NS = 16
NW = NC * NS
BPW = B // NW

RB = 2048


def _gather_body(idx_hbm, tbl_hbm, out_hbm, idx_v, rows_v, sem):
    wid = lax.axis_index("s") * NC + lax.axis_index("c")
    base = wid * BPW
    pltpu.sync_copy(idx_hbm.at[pl.ds(base, BPW)], idx_v)
    pltpu.async_copy(tbl_hbm.at[idx_v], rows_v, sem).wait()
    pltpu.sync_copy(rows_v, out_hbm.at[pl.ds(base, BPW)])


def _one_gather(idx, table):
    mesh = plsc.VectorSubcoreMesh(core_axis_name="c", subcore_axis_name="s")
    f = pl.kernel(
        _gather_body,
        mesh=mesh,
        compiler_params=pltpu.CompilerParams(use_tc_tiling_on_sc=False,
                                             has_side_effects=False,
                                             skip_device_barrier=True),
        out_type=jax.ShapeDtypeStruct((B, D), jnp.float32),
        scratch_types=[
            pltpu.VMEM((BPW,), jnp.int32),
            pltpu.VMEM((BPW, D), jnp.float32),
            pltpu.SemaphoreType.DMA,
        ],
    )
    return f(idx, table)


def _concat_body(u_ref, r_ref, g_ref, out_ref):
    out_ref[...] = jnp.concatenate([u_ref[...], r_ref[...], g_ref[...]],
                                   axis=1)


def kernel(uid, rid, ing, user_table, recipe_table, ingredient_table):
    u_emb = _one_gather(uid, user_table)
    r_emb = _one_gather(rid, recipe_table)
    i_emb = _one_gather(ing, ingredient_table)

    concat = pl.pallas_call(
        _concat_body,
        grid=(B // RB,),
        in_specs=[
            pl.BlockSpec((RB, D), lambda i: (i, 0)),
            pl.BlockSpec((RB, D), lambda i: (i, 0)),
            pl.BlockSpec((RB, D), lambda i: (i, 0)),
        ],
        out_specs=pl.BlockSpec((RB, 3 * D), lambda i: (i, 0)),
        out_shape=jax.ShapeDtypeStruct((B, 3 * D), jnp.float32),
    )
    return concat(u_emb, r_emb, i_emb)
